# baseline (speedup 1.0000x reference)
import functools

import jax
import jax.numpy as jnp
from jax import lax
from jax.experimental import pallas as pl
from jax.experimental.pallas import tpu as pltpu
from jax.experimental.pallas import tpu_sc as plsc

N = 10000
D = 128
B = 64
C = 10

NC = 2
NS = 16
NW = NC * NS
CHUNK = 312
BIGCHUNK = 320
NSMALL = 30
LANES = 16
DV = D // LANES


def _seg_body(x_hbm, b_hbm, sums_hbm, cnts_hbm, xv, bv, acc, cnt, semx, semb):
    wid = lax.axis_index("s") * NC + lax.axis_index("c")
    is_big = wid >= NSMALL
    base = jnp.where(is_big, NSMALL * CHUNK + (wid - NSMALL) * BIGCHUNK,
                     wid * CHUNK)

    def copies(nrows):
        cpx = pltpu.make_async_copy(x_hbm.at[pl.ds(base, nrows)],
                                    xv.at[pl.ds(0, nrows)], semx)
        cpb = pltpu.make_async_copy(b_hbm.at[pl.ds(base, nrows)],
                                    bv.at[pl.ds(0, nrows)], semb)
        return cpx, cpb

    @pl.when(jnp.logical_not(is_big))
    def _():
        cpx, cpb = copies(CHUNK)
        cpx.start()
        cpb.start()

    @pl.when(is_big)
    def _():
        cpx, cpb = copies(BIGCHUNK)
        cpx.start()
        cpb.start()

    zeros = jnp.zeros((LANES,), jnp.float32)

    def zero_row(r, _):
        for j in range(DV):
            acc[r, pl.ds(j * LANES, LANES)] = zeros
        cnt[r, :] = zeros
        return 0

    lax.fori_loop(0, B, zero_row, 0)

    def do_rows(r0, n, carry):
        cur, accs, cntf = carry
        segv = bv[pl.ds(r0, LANES)]
        for k in range(n):
            s = segv[k]
            fresh = s != cur
            keep = jnp.where(fresh, jnp.float32(0.0), jnp.float32(1.0))
            keepv = jnp.full((LANES,), keep)
            row = [xv[r0 + k, pl.ds(j * LANES, LANES)] for j in range(DV)]
            accs = tuple(row[j] + keepv * accs[j] for j in range(DV))
            cntf = 1.0 + keep * cntf
            for j in range(DV):
                acc[s, pl.ds(j * LANES, LANES)] = accs[j]
            cnt[s, :] = jnp.full((LANES,), cntf)
            cur = s
        return cur, accs, cntf

    def accumulate(nrows):
        carry0 = (jnp.int32(-1),
                  tuple(jnp.zeros((LANES,), jnp.float32) for _ in range(DV)),
                  jnp.float32(0.0))
        nfull = nrows // LANES
        tail = nrows - nfull * LANES

        def gbody(g, carry):
            return do_rows(g * LANES, LANES, carry)

        carry = lax.fori_loop(0, nfull, gbody, carry0)
        if tail:
            do_rows(nfull * LANES, tail, carry)

    @pl.when(jnp.logical_not(is_big))
    def _():
        cpx, cpb = copies(CHUNK)
        cpb.wait()
        cpx.wait()
        accumulate(CHUNK)

    @pl.when(is_big)
    def _():
        cpx, cpb = copies(BIGCHUNK)
        cpb.wait()
        cpx.wait()
        accumulate(BIGCHUNK)

    pltpu.sync_copy(acc, sums_hbm.at[wid])
    pltpu.sync_copy(cnt, cnts_hbm.at[wid])


@jax.jit
def _seg_pool(x, batch32):
    mesh = plsc.VectorSubcoreMesh(core_axis_name="c", subcore_axis_name="s")
    fn = functools.partial(
        pl.kernel,
        mesh=mesh,
        compiler_params=pltpu.CompilerParams(use_tc_tiling_on_sc=True),
        out_type=[
            jax.ShapeDtypeStruct((NW, B, D), jnp.float32),
            jax.ShapeDtypeStruct((NW, B, LANES), jnp.float32),
        ],
        scratch_types=[
            pltpu.VMEM((BIGCHUNK, D), jnp.float32),
            pltpu.VMEM((BIGCHUNK,), jnp.int32),
            pltpu.VMEM((B, D), jnp.float32),
            pltpu.VMEM((B, LANES), jnp.float32),
            pltpu.SemaphoreType.DMA,
            pltpu.SemaphoreType.DMA,
        ],
    )(_seg_body)
    return fn(x, batch32)


def _tail_body(sums_ref, cnts_ref, w_ref, b_ref, g_ref, beta_ref, o_ref):
    sums = jnp.sum(sums_ref[...], axis=0)
    counts = jnp.sum(cnts_ref[...], axis=0)[:, 0:1]
    mean = sums / jnp.clip(counts, 1.0, None)
    logits = jnp.dot(mean, w_ref[...].T,
                     preferred_element_type=jnp.float32) + b_ref[...]
    mu = jnp.mean(logits, axis=0, keepdims=True)
    var = jnp.mean((logits - mu) ** 2, axis=0, keepdims=True)
    o_ref[...] = (logits - mu) * lax.rsqrt(var + 1e-5) * g_ref[...] + beta_ref[...]


@jax.jit
def _tail(sums_p, cnts_p, W, b, gamma, beta):
    return pl.pallas_call(
        _tail_body,
        out_shape=jax.ShapeDtypeStruct((B, C), jnp.float32),
    )(sums_p, cnts_p, W, b.reshape(1, C), gamma.reshape(1, C), beta.reshape(1, C))


def kernel(x, edge_index, batch, coord, W, b, gamma, beta):
    del edge_index, coord
    batch32 = batch.astype(jnp.int32)
    sums_p, cnts_p = _seg_pool(x, batch32)
    return _tail(sums_p, cnts_p, W, b, gamma, beta)


# speedup vs baseline: 1.0064x; 1.0064x over previous
import functools

import jax
import jax.numpy as jnp
from jax import lax
from jax.experimental import pallas as pl
from jax.experimental.pallas import tpu as pltpu
from jax.experimental.pallas import tpu_sc as plsc

N = 10000
D = 128
B = 64
C = 10

NC = 2
NS = 16
NW = NC * NS
CHUNK = 312
BIGCHUNK = 320
NSMALL = 30
LANES = 16
DV = D // LANES


HALF = 160


def _seg_body(x_hbm, b_hbm, sums_hbm, cnts_hbm, xv, bv, acc, cnt,
              semx, semx2, semb):
    wid = lax.axis_index("s") * NC + lax.axis_index("c")
    is_big = wid >= NSMALL
    base = jnp.where(is_big, NSMALL * CHUNK + (wid - NSMALL) * BIGCHUNK,
                     wid * CHUNK)

    def copies(nrows):
        cpx = pltpu.make_async_copy(x_hbm.at[pl.ds(base, HALF)],
                                    xv.at[pl.ds(0, HALF)], semx)
        cpx2 = pltpu.make_async_copy(x_hbm.at[pl.ds(base + HALF, nrows - HALF)],
                                     xv.at[pl.ds(HALF, nrows - HALF)], semx2)
        cpb = pltpu.make_async_copy(b_hbm.at[pl.ds(base, nrows)],
                                    bv.at[pl.ds(0, nrows)], semb)
        return cpx, cpx2, cpb

    @pl.when(jnp.logical_not(is_big))
    def _():
        cpx, cpx2, cpb = copies(CHUNK)
        cpx.start()
        cpx2.start()
        cpb.start()

    @pl.when(is_big)
    def _():
        cpx, cpx2, cpb = copies(BIGCHUNK)
        cpx.start()
        cpx2.start()
        cpb.start()

    zeros = jnp.zeros((LANES,), jnp.float32)

    def zero_row(r, _):
        for j in range(DV):
            acc[r, pl.ds(j * LANES, LANES)] = zeros
        cnt[r, :] = zeros
        return 0

    lax.fori_loop(0, B, zero_row, 0)

    def do_rows(r0, n, carry):
        cur, accs, cntf = carry
        segv = bv[pl.ds(r0, LANES)]
        for k in range(n):
            s = segv[k]
            fresh = s != cur
            keep = jnp.where(fresh, jnp.float32(0.0), jnp.float32(1.0))
            keepv = jnp.full((LANES,), keep)
            row = [xv[r0 + k, pl.ds(j * LANES, LANES)] for j in range(DV)]
            accs = tuple(row[j] + keepv * accs[j] for j in range(DV))
            cntf = 1.0 + keep * cntf
            for j in range(DV):
                acc[s, pl.ds(j * LANES, LANES)] = accs[j]
            cnt[s, :] = jnp.full((LANES,), cntf)
            cur = s
        return cur, accs, cntf

    def gbody(g, carry):
        return do_rows(g * LANES, LANES, carry)

    def accumulate(nrows, cpx, cpx2, cpb):
        carry0 = (jnp.int32(-1),
                  tuple(jnp.zeros((LANES,), jnp.float32) for _ in range(DV)),
                  jnp.float32(0.0))
        nhalf = HALF // LANES
        nfull = nrows // LANES
        tail = nrows - nfull * LANES
        cpb.wait()
        cpx.wait()
        carry = lax.fori_loop(0, nhalf, gbody, carry0)
        cpx2.wait()
        carry = lax.fori_loop(nhalf, nfull, gbody, carry)
        if tail:
            do_rows(nfull * LANES, tail, carry)

    @pl.when(jnp.logical_not(is_big))
    def _():
        cpx, cpx2, cpb = copies(CHUNK)
        accumulate(CHUNK, cpx, cpx2, cpb)

    @pl.when(is_big)
    def _():
        cpx, cpx2, cpb = copies(BIGCHUNK)
        accumulate(BIGCHUNK, cpx, cpx2, cpb)

    pltpu.sync_copy(acc, sums_hbm.at[wid])
    pltpu.sync_copy(cnt, cnts_hbm.at[wid])


@jax.jit
def _seg_pool(x, batch32):
    mesh = plsc.VectorSubcoreMesh(core_axis_name="c", subcore_axis_name="s")
    fn = functools.partial(
        pl.kernel,
        mesh=mesh,
        compiler_params=pltpu.CompilerParams(use_tc_tiling_on_sc=True),
        out_type=[
            jax.ShapeDtypeStruct((NW, B, D), jnp.float32),
            jax.ShapeDtypeStruct((NW, B, LANES), jnp.float32),
        ],
        scratch_types=[
            pltpu.VMEM((BIGCHUNK, D), jnp.float32),
            pltpu.VMEM((BIGCHUNK,), jnp.int32),
            pltpu.VMEM((B, D), jnp.float32),
            pltpu.VMEM((B, LANES), jnp.float32),
            pltpu.SemaphoreType.DMA,
            pltpu.SemaphoreType.DMA,
            pltpu.SemaphoreType.DMA,
        ],
    )(_seg_body)
    return fn(x, batch32)


def _tail_body(sums_ref, cnts_ref, w_ref, b_ref, g_ref, beta_ref, o_ref):
    sums = jnp.sum(sums_ref[...], axis=0)
    counts = jnp.sum(cnts_ref[...], axis=0)[:, 0:1]
    mean = sums / jnp.clip(counts, 1.0, None)
    logits_t = lax.dot_general(w_ref[...], mean, (((1,), (1,)), ((), ())),
                               preferred_element_type=jnp.float32) + b_ref[...]
    mu = jnp.mean(logits_t, axis=1, keepdims=True)
    var = jnp.mean((logits_t - mu) ** 2, axis=1, keepdims=True)
    o_ref[...] = (logits_t - mu) * lax.rsqrt(var + 1e-5) * g_ref[...] + beta_ref[...]


@jax.jit
def _tail(sums_p, cnts_p, W, b, gamma, beta):
    out_t = pl.pallas_call(
        _tail_body,
        out_shape=jax.ShapeDtypeStruct((C, B), jnp.float32),
    )(sums_p, cnts_p, W, b.reshape(C, 1), gamma.reshape(C, 1), beta.reshape(C, 1))
    return out_t.T


def kernel(x, edge_index, batch, coord, W, b, gamma, beta):
    del edge_index, coord
    batch32 = batch.astype(jnp.int32)
    sums_p, cnts_p = _seg_pool(x, batch32)
    return _tail(sums_p, cnts_p, W, b, gamma, beta)


# speedup vs baseline: 1.0310x; 1.0245x over previous
import functools

import jax
import jax.numpy as jnp
from jax import lax
from jax.experimental import pallas as pl
from jax.experimental.pallas import tpu as pltpu
from jax.experimental.pallas import tpu_sc as plsc

N = 10000
D = 128
B = 64
C = 10

NC = 2
NS = 16
NW = NC * NS
CHUNK = 312
BIGCHUNK = 320
NSMALL = 30
LANES = 16
DV = D // LANES


HALF = 160


def _seg_body(x_hbm, b_hbm, sums_hbm, cnts_hbm, xv, bv, acc, cnt,
              semx, semx2, semb):
    wid = lax.axis_index("s") * NC + lax.axis_index("c")
    is_big = wid >= NSMALL
    base = jnp.where(is_big, NSMALL * CHUNK + (wid - NSMALL) * BIGCHUNK,
                     wid * CHUNK)

    def copies(nrows):
        cpx = pltpu.make_async_copy(x_hbm.at[pl.ds(base, HALF)],
                                    xv.at[pl.ds(0, HALF)], semx)
        cpx2 = pltpu.make_async_copy(x_hbm.at[pl.ds(base + HALF, nrows - HALF)],
                                     xv.at[pl.ds(HALF, nrows - HALF)], semx2)
        cpb = pltpu.make_async_copy(b_hbm.at[pl.ds(base, nrows)],
                                    bv.at[pl.ds(0, nrows)], semb)
        return cpx, cpx2, cpb

    @pl.when(jnp.logical_not(is_big))
    def _():
        cpx, cpx2, cpb = copies(CHUNK)
        cpx.start()
        cpx2.start()
        cpb.start()

    @pl.when(is_big)
    def _():
        cpx, cpx2, cpb = copies(BIGCHUNK)
        cpx.start()
        cpx2.start()
        cpb.start()

    zeros = jnp.zeros((LANES,), jnp.float32)

    def zero_row(r, _):
        for j in range(DV):
            acc[r, pl.ds(j * LANES, LANES)] = zeros
        return 0

    lax.fori_loop(0, B, zero_row, 0)
    for r in range((B + LANES) // LANES):
        cnt[pl.ds(r * LANES, LANES)] = zeros

    ones = jnp.ones((LANES,), jnp.float32)

    def do_rows(r0, n, carry):
        cur, accs = carry
        segv = bv[pl.ds(r0, LANES)]
        if n == LANES:
            plsc.addupdate_scatter(cnt, [segv], ones)
        else:
            valid = jnp.clip(n - lax.iota(jnp.int32, LANES), 0, 1)
            segv_t = segv * valid + (1 - valid) * B
            plsc.addupdate_scatter(cnt, [segv_t], ones)
        for k in range(n):
            s = segv[k]
            fresh = s != cur
            keep = jnp.where(fresh, jnp.float32(0.0), jnp.float32(1.0))
            keepv = jnp.full((LANES,), keep)
            row = [xv[r0 + k, pl.ds(j * LANES, LANES)] for j in range(DV)]
            accs = tuple(row[j] + keepv * accs[j] for j in range(DV))
            for j in range(DV):
                acc[s, pl.ds(j * LANES, LANES)] = accs[j]
            cur = s
        return cur, accs

    def gbody(g, carry):
        return do_rows(g * LANES, LANES, carry)

    def accumulate(nrows, cpx, cpx2, cpb):
        carry0 = (jnp.int32(-1),
                  tuple(jnp.zeros((LANES,), jnp.float32) for _ in range(DV)))
        nhalf = HALF // LANES
        nfull = nrows // LANES
        tail = nrows - nfull * LANES
        cpb.wait()
        cpx.wait()
        carry = lax.fori_loop(0, nhalf, gbody, carry0)
        cpx2.wait()
        carry = lax.fori_loop(nhalf, nfull, gbody, carry)
        if tail:
            do_rows(nfull * LANES, tail, carry)

    @pl.when(jnp.logical_not(is_big))
    def _():
        cpx, cpx2, cpb = copies(CHUNK)
        accumulate(CHUNK, cpx, cpx2, cpb)

    @pl.when(is_big)
    def _():
        cpx, cpx2, cpb = copies(BIGCHUNK)
        accumulate(BIGCHUNK, cpx, cpx2, cpb)

    pltpu.sync_copy(acc, sums_hbm.at[wid])
    pltpu.sync_copy(cnt, cnts_hbm.at[wid])


@jax.jit
def _seg_pool(x, batch32):
    mesh = plsc.VectorSubcoreMesh(core_axis_name="c", subcore_axis_name="s")
    fn = functools.partial(
        pl.kernel,
        mesh=mesh,
        compiler_params=pltpu.CompilerParams(needs_layout_passes=False),
        out_type=[
            jax.ShapeDtypeStruct((NW, B, D), jnp.float32),
            jax.ShapeDtypeStruct((NW, B + LANES), jnp.float32),
        ],
        scratch_types=[
            pltpu.VMEM((BIGCHUNK, D), jnp.float32),
            pltpu.VMEM((BIGCHUNK,), jnp.int32),
            pltpu.VMEM((B, D), jnp.float32),
            pltpu.VMEM((B + LANES,), jnp.float32),
            pltpu.SemaphoreType.DMA,
            pltpu.SemaphoreType.DMA,
            pltpu.SemaphoreType.DMA,
        ],
    )(_seg_body)
    return fn(x, batch32)


def _tail_body(sums_ref, cnts_ref, w_ref, b_ref, g_ref, beta_ref, o_ref):
    sums = jnp.sum(sums_ref[...], axis=0)
    counts = jnp.sum(cnts_ref[...], axis=0)[:B, None]
    mean = sums / jnp.clip(counts, 1.0, None)
    logits_t = lax.dot_general(w_ref[...], mean, (((1,), (1,)), ((), ())),
                               preferred_element_type=jnp.float32) + b_ref[...]
    mu = jnp.mean(logits_t, axis=1, keepdims=True)
    var = jnp.mean((logits_t - mu) ** 2, axis=1, keepdims=True)
    o_ref[...] = (logits_t - mu) * lax.rsqrt(var + 1e-5) * g_ref[...] + beta_ref[...]


@jax.jit
def _tail(sums_p, cnts_p, W, b, gamma, beta):
    out_t = pl.pallas_call(
        _tail_body,
        out_shape=jax.ShapeDtypeStruct((C, B), jnp.float32),
    )(sums_p, cnts_p, W, b.reshape(C, 1), gamma.reshape(C, 1), beta.reshape(C, 1))
    return out_t.T


def kernel(x, edge_index, batch, coord, W, b, gamma, beta):
    del edge_index, coord
    batch32 = batch.astype(jnp.int32)
    sums_p, cnts_p = _seg_pool(x, batch32)
    return _tail(sums_p, cnts_p, W, b, gamma, beta)
